# baseline (speedup 1.0000x reference)
import functools

import jax
import jax.numpy as jnp
from jax import lax
from jax.experimental import pallas as pl
from jax.experimental.pallas import tpu as pltpu
from jax.experimental.pallas import tpu_sc as plsc

NUM_FEATURES = 8
VOCAB = 100000
D = 64
B = 4096
BAG = 20

NC = 2
NS = 16
NW = NC * NS
BPW = B // NW
LANES = 16
NG = D // LANES
NBG = BPW // LANES

TCB = 8192
TGRID = -(-VOCAB // (2 * TCB))
VPAD = TGRID * 2 * TCB


def _tpose_body(wt_ref, out_ref):
    a = wt_ref[:, pl.ds(0, TCB)].T
    b = wt_ref[:, pl.ds(TCB, TCB)].T
    out_ref[...] = jnp.concatenate([a, b], axis=1)


def _tpose(wt):
    return pl.pallas_call(
        _tpose_body,
        grid=(TGRID,),
        in_specs=[pl.BlockSpec((D, 2 * TCB), lambda g: (0, g))],
        out_specs=pl.BlockSpec((TCB, 2 * D), lambda g: (g, 0)),
        out_shape=jax.ShapeDtypeStruct((VPAD // 2, 2 * D), jnp.float32),
    )(wt)


def _make_kernel():
    mesh = plsc.VectorSubcoreMesh(core_axis_name="c", subcore_axis_name="s")

    @functools.partial(
        pl.kernel,
        mesh=mesh,
        compiler_params=pltpu.CompilerParams(
            use_tc_tiling_on_sc=False, needs_layout_passes=False,
            skip_device_barrier=True),
        out_type=jax.ShapeDtypeStruct((B, D), jnp.float32),
        scratch_types=[
            pltpu.VMEM((BPW, BAG), jnp.int32),
            pltpu.VMEM((BAG, BPW), jnp.int32),
            pltpu.VMEM((BPW, D), jnp.float32),
            pltpu.VMEM((BPW, D), jnp.float32),
            pltpu.SemaphoreType.DMA,
            pltpu.SemaphoreType.DMA,
        ],
    )
    def ebag1(idx_hbm, W_hbm, out_hbm, idx_v, idxT_v, acc_v, out_v,
              gsem, osem):
        wid = lax.axis_index("s") * NC + lax.axis_index("c")
        inv = jnp.full((LANES,), 1.0 / BAG, dtype=jnp.float32)
        zero = jnp.zeros((LANES,), dtype=jnp.float32)
        lane_iota = lax.iota(jnp.int32, LANES)

        pltpu.sync_copy(idx_hbm.at[pl.ds(wid * BPW, BPW), :], idx_v)

        def trans_body(j, _):
            col = jnp.full((LANES,), 0, jnp.int32) + j
            for grp in range(NBG):
                rows = lane_iota + grp * LANES
                v = plsc.load_gather(idx_v, [rows, col])
                v = (
                    ((v >> 14) << 14)
                    + ((v & (TCB - 1)) << 1)
                    + ((v >> 13) & 1)
                )
                idxT_v[j, pl.ds(grp * LANES, LANES)] = v
            return 0
        lax.fori_loop(0, BAG, trans_body, 0)

        def zero_body(b, _):
            for g in range(NG):
                acc_v[b, pl.ds(g * LANES, LANES)] = zero
            return 0
        lax.fori_loop(0, BPW, zero_body, 0)

        copies = [
            pltpu.async_copy(
                W_hbm.at[idxT_v.at[j]], acc_v, gsem, add=True)
            for j in range(BAG)
        ]
        for cp in copies:
            cp.wait()

        def scale_body(b, _):
            for g in range(NG):
                sl = pl.ds(g * LANES, LANES)
                out_v[b, sl] = acc_v[b, sl] * inv
            return 0
        lax.fori_loop(0, BPW, scale_body, 0)

        pltpu.async_copy(
            out_v, out_hbm.at[pl.ds(wid * BPW, BPW), :], osem).wait()

    return ebag1


_EBAG1 = _make_kernel()


def kernel(feat_0, feat_1, feat_2, feat_3, feat_4, feat_5, feat_6, feat_7,
           W_0, W_1, W_2, W_3, W_4, W_5, W_6, W_7):
    feats = (feat_0, feat_1, feat_2, feat_3, feat_4, feat_5, feat_6, feat_7)
    Ws = (W_0, W_1, W_2, W_3, W_4, W_5, W_6, W_7)
    outs = []
    for f, w in zip(feats, Ws):
        w_lin = _tpose(w.T).reshape(VPAD, D)
        outs.append(_EBAG1(f, w_lin))
    return tuple(outs)


# speedup vs baseline: 1.0012x; 1.0012x over previous
import functools

import jax
import jax.numpy as jnp
from jax import lax
from jax.experimental import pallas as pl
from jax.experimental.pallas import tpu as pltpu
from jax.experimental.pallas import tpu_sc as plsc

NUM_FEATURES = 8
VOCAB = 100000
D = 64
B = 4096
BAG = 20

NC = 2
NS = 16
NW = NC * NS
BPW = B // NW
LANES = 16
NG = D // LANES
NBG = BPW // LANES

TCB = 4096
TGRID = -(-VOCAB // (2 * TCB))
VPAD = TGRID * 2 * TCB


def _tpose_body(wt_ref, out_ref):
    a = wt_ref[:, pl.ds(0, TCB)].T
    b = wt_ref[:, pl.ds(TCB, TCB)].T
    out_ref[...] = jnp.concatenate([a, b], axis=1)


def _tpose(wt):
    return pl.pallas_call(
        _tpose_body,
        grid=(TGRID,),
        in_specs=[pl.BlockSpec((D, 2 * TCB), lambda g: (0, g))],
        out_specs=pl.BlockSpec((TCB, 2 * D), lambda g: (g, 0)),
        out_shape=jax.ShapeDtypeStruct((VPAD // 2, 2 * D), jnp.float32),
    )(wt)


def _make_kernel():
    mesh = plsc.VectorSubcoreMesh(core_axis_name="c", subcore_axis_name="s")

    @functools.partial(
        pl.kernel,
        mesh=mesh,
        compiler_params=pltpu.CompilerParams(
            use_tc_tiling_on_sc=False, needs_layout_passes=False,
            skip_device_barrier=True),
        out_type=jax.ShapeDtypeStruct((B, D), jnp.float32),
        scratch_types=[
            pltpu.VMEM((BPW, BAG), jnp.int32),
            pltpu.VMEM((BAG, BPW), jnp.int32),
            pltpu.VMEM((BPW, D), jnp.float32),
            pltpu.VMEM((BPW, D), jnp.float32),
            pltpu.SemaphoreType.DMA,
            pltpu.SemaphoreType.DMA,
        ],
    )
    def ebag1(idx_hbm, W_hbm, out_hbm, idx_v, idxT_v, acc_v, out_v,
              gsem, osem):
        wid = lax.axis_index("s") * NC + lax.axis_index("c")
        inv = jnp.full((LANES,), 1.0 / BAG, dtype=jnp.float32)
        zero = jnp.zeros((LANES,), dtype=jnp.float32)
        lane_iota = lax.iota(jnp.int32, LANES)

        pltpu.sync_copy(idx_hbm.at[pl.ds(wid * BPW, BPW), :], idx_v)

        def trans_body(j, _):
            col = jnp.full((LANES,), 0, jnp.int32) + j
            for grp in range(NBG):
                rows = lane_iota + grp * LANES
                v = plsc.load_gather(idx_v, [rows, col])
                v = (
                    ((v >> 13) << 13)
                    + ((v & (TCB - 1)) << 1)
                    + ((v >> 12) & 1)
                )
                idxT_v[j, pl.ds(grp * LANES, LANES)] = v
            return 0
        lax.fori_loop(0, BAG, trans_body, 0)

        def zero_body(b, _):
            for g in range(NG):
                acc_v[b, pl.ds(g * LANES, LANES)] = zero
            return 0
        lax.fori_loop(0, BPW, zero_body, 0)

        copies = [
            pltpu.async_copy(
                W_hbm.at[idxT_v.at[j]], acc_v, gsem, add=True)
            for j in range(BAG)
        ]
        for cp in copies:
            cp.wait()

        def scale_body(b, _):
            for g in range(NG):
                sl = pl.ds(g * LANES, LANES)
                out_v[b, sl] = acc_v[b, sl] * inv
            return 0
        lax.fori_loop(0, BPW, scale_body, 0)

        pltpu.async_copy(
            out_v, out_hbm.at[pl.ds(wid * BPW, BPW), :], osem).wait()

    return ebag1


_EBAG1 = _make_kernel()


def kernel(feat_0, feat_1, feat_2, feat_3, feat_4, feat_5, feat_6, feat_7,
           W_0, W_1, W_2, W_3, W_4, W_5, W_6, W_7):
    feats = (feat_0, feat_1, feat_2, feat_3, feat_4, feat_5, feat_6, feat_7)
    Ws = (W_0, W_1, W_2, W_3, W_4, W_5, W_6, W_7)
    outs = []
    for f, w in zip(feats, Ws):
        w_lin = _tpose(w.T).reshape(VPAD, D)
        outs.append(_EBAG1(f, w_lin))
    return tuple(outs)
